# baseline (speedup 1.0000x reference)
import functools

import jax
import jax.numpy as jnp
from jax import lax
from jax.experimental import pallas as pl
from jax.experimental.pallas import tpu as pltpu
from jax.experimental.pallas import tpu_sc as plsc

_B_ROWS = 16384
_SEQ = 20
_DIM = 64
_N_IDX = _B_ROWS * _SEQ
_VOCAB = 1000000

_NC = 2
_NS = 16
_NW = _NC * _NS

_BLK = 128
_NBLK_FULL = _VOCAB // _BLK
_TAIL = _VOCAB - _NBLK_FULL * _BLK
_BLK_PER_TILE = _NBLK_FULL // _NW
_BLK_REM = _NBLK_FULL - _BLK_PER_TILE * _NW

_IDXW = 128
_ROWS_PER_W = _N_IDX // _NW
_IDX_ROWS_PER_W = _ROWS_PER_W // _IDXW
_CHUNK = _IDXW
_N_CHUNKS = _ROWS_PER_W // _CHUNK
_NBUF = 5


def _make_transpose():
  mesh = plsc.VectorSubcoreMesh(core_axis_name="c", subcore_axis_name="s")

  @functools.partial(
      pl.kernel,
      out_type=jax.ShapeDtypeStruct((_VOCAB // 2, 128), jnp.float32),
      mesh=mesh,
      scratch_types=[
          pltpu.VMEM((2, 8, 8, 128), jnp.float32),
          pltpu.VMEM((2, 64, 128), jnp.float32),
          pltpu.SemaphoreType.DMA,
          pltpu.SemaphoreType.DMA,
          pltpu.SemaphoreType.DMA,
          pltpu.SemaphoreType.DMA,
      ],
      compiler_params=pltpu.CompilerParams(use_tc_tiling_on_sc=True,
                                           needs_layout_passes=False),
  )
  def transpose_kernel(wt_hbm, tail_hbm, out_hbm, bbuf, tbuf, sb0, sb1,
                       st0, st1):
    sem_b = (sb0, sb1)
    sem_t = (st0, st1)
    wid = lax.axis_index("s") * _NC + lax.axis_index("c")

    iota = lax.iota(jnp.int32, 16)
    jr_of = iota // 8
    s_of = iota % 8

    def load_copies(blk, par):
      return [
          pltpu.make_async_copy(
              wt_hbm.at[pl.ds(8 * jr, 8), pl.ds(blk * _BLK, _BLK)],
              bbuf.at[par].at[jr],
              sem_b[par],
          )
          for jr in range(8)
      ]

    def store_copy(blk, par):
      return pltpu.make_async_copy(
          tbuf.at[par],
          out_hbm.at[pl.ds(blk * 64, 64)],
          sem_t[par],
      )

    def transpose_block(par):
      b_ref = bbuf.at[par]
      t_ref = tbuf.at[par]

      for r in range(64):
        for h in range(2):
          lvec = jnp.full((16,), 2 * r + h, jnp.int32)
          for g in range(4):
            v = plsc.load_gather(b_ref, [2 * g + jr_of, s_of, lvec])
            t_ref[r, pl.ds(h * 64 + 16 * g, 16)] = v

    def blk_of(t, phase):
      return wid + _NW * (2 * t + phase)

    for cp in load_copies(blk_of(0, 0), 0):
      cp.start()
    for cp in load_copies(blk_of(0, 1), 1):
      cp.start()

    def super_body(t, carry):
      for par in range(2):
        blk = blk_of(t, par)
        for cp in load_copies(blk, par):
          cp.wait()

        @pl.when(t >= 1)
        def _():
          store_copy(blk - 2 * _NW, par).wait()

        transpose_block(par)
        store_copy(blk, par).start()

        @pl.when(t < _BLK_PER_TILE // 2 - 1)
        def _():
          for cp in load_copies(blk + 2 * _NW, par):
            cp.start()
      return carry

    lax.fori_loop(0, _BLK_PER_TILE // 2, super_body, 0, unroll=False)

    last_t = _BLK_PER_TILE // 2 - 1
    store_copy(blk_of(last_t, 0), 0).wait()
    store_copy(blk_of(last_t, 1), 1).wait()

    @pl.when(wid < _BLK_REM)
    def _():
      blk = _BLK_PER_TILE * _NW + wid
      for cp in load_copies(blk, 0):
        cp.start()
      for cp in load_copies(blk, 0):
        cp.wait()
      transpose_block(0)
      store_copy(blk, 0).start()
      store_copy(blk, 0).wait()

    @pl.when(wid == _BLK_REM)
    def _():
      pltpu.sync_copy(tail_hbm, tbuf.at[0].at[pl.ds(0, 32)])
      pltpu.sync_copy(tbuf.at[0].at[pl.ds(0, 32)],
                      out_hbm.at[pl.ds(_NBLK_FULL * 64, 32)])

  return transpose_kernel


def _make_gather():
  mesh = plsc.VectorSubcoreMesh(core_axis_name="c", subcore_axis_name="s")

  @functools.partial(
      pl.kernel,
      out_type=jax.ShapeDtypeStruct((_N_IDX, _DIM), jnp.float32),
      mesh=mesh,
      scratch_types=(
          [pltpu.VMEM((_IDX_ROWS_PER_W, _IDXW), jnp.int32),
           pltpu.VMEM((_NBUF, _CHUNK, _DIM), jnp.float32)]
          + [pltpu.SemaphoreType.DMA] * (2 * _NBUF)
      ),
      compiler_params=pltpu.CompilerParams(use_tc_tiling_on_sc=False),
  )
  def gather_kernel(table_hbm, idx_hbm, out_hbm, idx_v, rows_v, *sems):
    sem_g = sems[:_NBUF]
    sem_o = sems[_NBUF:]
    wid = lax.axis_index("s") * _NC + lax.axis_index("c")
    idx_row_base = wid * _IDX_ROWS_PER_W
    out_base = wid * _ROWS_PER_W

    pltpu.sync_copy(idx_hbm.at[pl.ds(idx_row_base, _IDX_ROWS_PER_W)], idx_v)

    def g_copy(c, b):
      return pltpu.make_async_copy(
          table_hbm.at[idx_v.at[c]], rows_v.at[b], sem_g[b])

    def o_copy(c, b):
      return pltpu.make_async_copy(
          rows_v.at[b],
          out_hbm.at[pl.ds(out_base + c * _CHUNK, _CHUNK)],
          sem_o[b],
      )

    for b in range(_NBUF):
      g_copy(b, b).start()

    def super_body(s, carry):
      c0 = s * _NBUF
      for b in range(_NBUF):
        c = c0 + b
        g_copy(c, b).wait()
        o_copy(c, b).start()
        pb = (b - 1) % _NBUF
        cprev = c - 1
        nxt = cprev + _NBUF

        @pl.when(jnp.logical_and(cprev >= 0, nxt < _N_CHUNKS))
        def _():
          o_copy(cprev, pb).wait()
          for cp in [g_copy(nxt, pb)]:
            cp.start()

      return carry

    lax.fori_loop(0, _N_CHUNKS // _NBUF, super_body, 0, unroll=False)

    for b in range(_NBUF):
      o_copy(_N_CHUNKS - _NBUF + b, b).wait()

  return gather_kernel


_transpose = _make_transpose()
_gather = _make_gather()


def kernel(input_, weight):
  idx = input_.reshape(-1).astype(jnp.int32).reshape(_N_IDX // _IDXW, _IDXW)
  tail = weight[_NBLK_FULL * _BLK:].reshape(32, 128)
  table2 = _transpose(weight.T, tail)
  table = table2.reshape(_VOCAB, _DIM)
  out = _gather(table, idx)
  return out.reshape(_B_ROWS, _SEQ, _DIM)


# speedup vs baseline: 1.7331x; 1.7331x over previous
import functools

import jax
import jax.numpy as jnp
from jax import lax
from jax.experimental import pallas as pl
from jax.experimental.pallas import tpu as pltpu
from jax.experimental.pallas import tpu_sc as plsc

_B_ROWS = 16384
_SEQ = 20
_DIM = 64
_N_IDX = _B_ROWS * _SEQ
_VOCAB = 1000000

_NC = 2
_NS = 16
_NW = _NC * _NS

_BLK = 128
_NBLK_FULL = _VOCAB // _BLK
_TAIL = _VOCAB - _NBLK_FULL * _BLK
_BLK_PER_TILE = _NBLK_FULL // _NW
_BLK_REM = _NBLK_FULL - _BLK_PER_TILE * _NW

_IDXW = 128
_ROWS_PER_W = _N_IDX // _NW
_IDX_ROWS_PER_W = _ROWS_PER_W // _IDXW
_CHUNK = _IDXW
_N_CHUNKS = _ROWS_PER_W // _CHUNK
_NBUF = 5


def _make_transpose():
  mesh = plsc.VectorSubcoreMesh(core_axis_name="c", subcore_axis_name="s")

  @functools.partial(
      pl.kernel,
      out_type=jax.ShapeDtypeStruct((_VOCAB // 2, 128), jnp.float32),
      mesh=mesh,
      scratch_types=[
          pltpu.VMEM((2, 8, 8, 128), jnp.float32),
          pltpu.VMEM((2, 64, 128), jnp.float32),
          pltpu.SemaphoreType.DMA,
          pltpu.SemaphoreType.DMA,
          pltpu.SemaphoreType.DMA,
          pltpu.SemaphoreType.DMA,
      ],
      compiler_params=pltpu.CompilerParams(use_tc_tiling_on_sc=True,
                                           needs_layout_passes=False),
  )
  def transpose_kernel(wt_hbm, tail_hbm, out_hbm, bbuf, tbuf, sb0, sb1,
                       st0, st1):
    sem_b = (sb0, sb1)
    sem_t = (st0, st1)
    wid = lax.axis_index("s") * _NC + lax.axis_index("c")

    iota = lax.iota(jnp.int32, 16)
    jr_of = iota // 8
    s_of = iota % 8

    def load_copies(blk, par):
      return [
          pltpu.make_async_copy(
              wt_hbm.at[pl.ds(8 * jr, 8), pl.ds(blk * _BLK, _BLK)],
              bbuf.at[par].at[jr],
              sem_b[par],
          )
          for jr in range(8)
      ]

    def store_copy(blk, par):
      return pltpu.make_async_copy(
          tbuf.at[par],
          out_hbm.at[pl.ds(blk * 64, 64)],
          sem_t[par],
      )

    def transpose_block(par):
      b_ref = bbuf.at[par]
      t_ref = tbuf.at[par]

      @plsc.parallel_loop(0, 64, unroll=4)
      def _(r):
        for h in range(2):
          lvec = jnp.full((16,), 0, jnp.int32) + (2 * r + h)
          for g in range(4):
            v = plsc.load_gather(b_ref, [2 * g + jr_of, s_of, lvec])
            t_ref[r, pl.ds(h * 64 + 16 * g, 16)] = v

    def blk_of(t, phase):
      return wid + _NW * (2 * t + phase)

    for cp in load_copies(blk_of(0, 0), 0):
      cp.start()
    for cp in load_copies(blk_of(0, 1), 1):
      cp.start()

    def super_body(t, carry):
      for par in range(2):
        blk = blk_of(t, par)
        for cp in load_copies(blk, par):
          cp.wait()

        @pl.when(t >= 1)
        def _():
          store_copy(blk - 2 * _NW, par).wait()

        transpose_block(par)
        store_copy(blk, par).start()

        @pl.when(t < _BLK_PER_TILE // 2 - 1)
        def _():
          for cp in load_copies(blk + 2 * _NW, par):
            cp.start()
      return carry

    lax.fori_loop(0, _BLK_PER_TILE // 2, super_body, 0, unroll=False)

    last_t = _BLK_PER_TILE // 2 - 1
    store_copy(blk_of(last_t, 0), 0).wait()
    store_copy(blk_of(last_t, 1), 1).wait()

    @pl.when(wid < _BLK_REM)
    def _():
      blk = _BLK_PER_TILE * _NW + wid
      for cp in load_copies(blk, 0):
        cp.start()
      for cp in load_copies(blk, 0):
        cp.wait()
      transpose_block(0)
      store_copy(blk, 0).start()
      store_copy(blk, 0).wait()

    @pl.when(wid == _BLK_REM)
    def _():
      pltpu.sync_copy(tail_hbm, tbuf.at[0].at[pl.ds(0, 32)])
      pltpu.sync_copy(tbuf.at[0].at[pl.ds(0, 32)],
                      out_hbm.at[pl.ds(_NBLK_FULL * 64, 32)])

  return transpose_kernel


def _make_gather():
  mesh = plsc.VectorSubcoreMesh(core_axis_name="c", subcore_axis_name="s")

  @functools.partial(
      pl.kernel,
      out_type=jax.ShapeDtypeStruct((_N_IDX, _DIM), jnp.float32),
      mesh=mesh,
      scratch_types=(
          [pltpu.VMEM((_IDX_ROWS_PER_W, _IDXW), jnp.int32),
           pltpu.VMEM((_NBUF, _CHUNK, _DIM), jnp.float32)]
          + [pltpu.SemaphoreType.DMA] * (2 * _NBUF)
      ),
      compiler_params=pltpu.CompilerParams(use_tc_tiling_on_sc=False),
  )
  def gather_kernel(table_hbm, idx_hbm, out_hbm, idx_v, rows_v, *sems):
    sem_g = sems[:_NBUF]
    sem_o = sems[_NBUF:]
    wid = lax.axis_index("s") * _NC + lax.axis_index("c")
    idx_row_base = wid * _IDX_ROWS_PER_W
    out_base = wid * _ROWS_PER_W

    pltpu.sync_copy(idx_hbm.at[pl.ds(idx_row_base, _IDX_ROWS_PER_W)], idx_v)

    def g_copy(c, b):
      return pltpu.make_async_copy(
          table_hbm.at[idx_v.at[c]], rows_v.at[b], sem_g[b])

    def o_copy(c, b):
      return pltpu.make_async_copy(
          rows_v.at[b],
          out_hbm.at[pl.ds(out_base + c * _CHUNK, _CHUNK)],
          sem_o[b],
      )

    for b in range(_NBUF):
      g_copy(b, b).start()

    def super_body(s, carry):
      c0 = s * _NBUF
      for b in range(_NBUF):
        c = c0 + b
        g_copy(c, b).wait()
        o_copy(c, b).start()
        pb = (b - 1) % _NBUF
        cprev = c - 1
        nxt = cprev + _NBUF

        @pl.when(jnp.logical_and(cprev >= 0, nxt < _N_CHUNKS))
        def _():
          o_copy(cprev, pb).wait()
          for cp in [g_copy(nxt, pb)]:
            cp.start()

      return carry

    lax.fori_loop(0, _N_CHUNKS // _NBUF, super_body, 0, unroll=False)

    for b in range(_NBUF):
      o_copy(_N_CHUNKS - _NBUF + b, b).wait()

  return gather_kernel


_transpose = _make_transpose()
_gather = _make_gather()


def kernel(input_, weight):
  idx = input_.reshape(-1).astype(jnp.int32).reshape(_N_IDX // _IDXW, _IDXW)
  tail = weight[_NBLK_FULL * _BLK:].reshape(32, 128)
  table2 = _transpose(weight.T, tail)
  table = table2.reshape(_VOCAB, _DIM)
  out = _gather(table, idx)
  return out.reshape(_B_ROWS, _SEQ, _DIM)


# speedup vs baseline: 1.7338x; 1.0004x over previous
import functools

import jax
import jax.numpy as jnp
from jax import lax
from jax.experimental import pallas as pl
from jax.experimental.pallas import tpu as pltpu
from jax.experimental.pallas import tpu_sc as plsc

_B_ROWS = 16384
_SEQ = 20
_DIM = 64
_N_IDX = _B_ROWS * _SEQ
_VOCAB = 1000000

_NC = 2
_NS = 16
_NW = _NC * _NS

_BLK = 128
_NBLK_FULL = _VOCAB // _BLK
_TAIL = _VOCAB - _NBLK_FULL * _BLK
_BLK_PER_TILE = _NBLK_FULL // _NW
_BLK_REM = _NBLK_FULL - _BLK_PER_TILE * _NW

_IDXW = 128
_ROWS_PER_W = _N_IDX // _NW
_IDX_ROWS_PER_W = _ROWS_PER_W // _IDXW
_CHUNK = _IDXW
_N_CHUNKS = _ROWS_PER_W // _CHUNK
_NBUF = 5


def _make_transpose():
  mesh = plsc.VectorSubcoreMesh(core_axis_name="c", subcore_axis_name="s")

  @functools.partial(
      pl.kernel,
      out_type=jax.ShapeDtypeStruct((_VOCAB // 2, 128), jnp.float32),
      mesh=mesh,
      scratch_types=[
          pltpu.VMEM((2, 8, 8, 128), jnp.float32),
          pltpu.VMEM((2, 64, 128), jnp.float32),
          pltpu.SemaphoreType.DMA,
          pltpu.SemaphoreType.DMA,
          pltpu.SemaphoreType.DMA,
          pltpu.SemaphoreType.DMA,
      ],
      compiler_params=pltpu.CompilerParams(use_tc_tiling_on_sc=True,
                                           needs_layout_passes=False),
  )
  def transpose_kernel(wt_hbm, tail_hbm, out_hbm, bbuf, tbuf, sb0, sb1,
                       st0, st1):
    sem_b = (sb0, sb1)
    sem_t = (st0, st1)
    wid = lax.axis_index("s") * _NC + lax.axis_index("c")

    iota = lax.iota(jnp.int32, 16)
    jr_of = iota // 8
    s_of = iota % 8

    def load_copies(blk, par):
      return [
          pltpu.make_async_copy(
              wt_hbm.at[pl.ds(8 * jr, 8), pl.ds(blk * _BLK, _BLK)],
              bbuf.at[par].at[jr],
              sem_b[par],
          )
          for jr in range(8)
      ]

    def store_copy(blk, par):
      return pltpu.make_async_copy(
          tbuf.at[par],
          out_hbm.at[pl.ds(blk * 64, 64)],
          sem_t[par],
      )

    def transpose_block(par):
      b_ref = bbuf.at[par]
      t_ref = tbuf.at[par]

      @plsc.parallel_loop(0, 64, unroll=8)
      def _(r):
        for h in range(2):
          lvec = jnp.full((16,), 0, jnp.int32) + (2 * r + h)
          for g in range(4):
            v = plsc.load_gather(b_ref, [2 * g + jr_of, s_of, lvec])
            t_ref[r, pl.ds(h * 64 + 16 * g, 16)] = v

    def blk_of(t, phase):
      return wid + _NW * (2 * t + phase)

    for cp in load_copies(blk_of(0, 0), 0):
      cp.start()
    for cp in load_copies(blk_of(0, 1), 1):
      cp.start()

    def super_body(t, carry):
      for par in range(2):
        blk = blk_of(t, par)
        for cp in load_copies(blk, par):
          cp.wait()

        @pl.when(t >= 1)
        def _():
          store_copy(blk - 2 * _NW, par).wait()

        transpose_block(par)
        store_copy(blk, par).start()

        @pl.when(t < _BLK_PER_TILE // 2 - 1)
        def _():
          for cp in load_copies(blk + 2 * _NW, par):
            cp.start()
      return carry

    lax.fori_loop(0, _BLK_PER_TILE // 2, super_body, 0, unroll=False)

    last_t = _BLK_PER_TILE // 2 - 1
    store_copy(blk_of(last_t, 0), 0).wait()
    store_copy(blk_of(last_t, 1), 1).wait()

    @pl.when(wid < _BLK_REM)
    def _():
      blk = _BLK_PER_TILE * _NW + wid
      for cp in load_copies(blk, 0):
        cp.start()
      for cp in load_copies(blk, 0):
        cp.wait()
      transpose_block(0)
      store_copy(blk, 0).start()
      store_copy(blk, 0).wait()

    @pl.when(wid == _BLK_REM)
    def _():
      pltpu.sync_copy(tail_hbm, tbuf.at[0].at[pl.ds(0, 32)])
      pltpu.sync_copy(tbuf.at[0].at[pl.ds(0, 32)],
                      out_hbm.at[pl.ds(_NBLK_FULL * 64, 32)])

  return transpose_kernel


def _make_gather():
  mesh = plsc.VectorSubcoreMesh(core_axis_name="c", subcore_axis_name="s")

  @functools.partial(
      pl.kernel,
      out_type=jax.ShapeDtypeStruct((_N_IDX, _DIM), jnp.float32),
      mesh=mesh,
      scratch_types=(
          [pltpu.VMEM((_IDX_ROWS_PER_W, _IDXW), jnp.int32),
           pltpu.VMEM((_NBUF, _CHUNK, _DIM), jnp.float32)]
          + [pltpu.SemaphoreType.DMA] * (2 * _NBUF)
      ),
      compiler_params=pltpu.CompilerParams(use_tc_tiling_on_sc=False),
  )
  def gather_kernel(table_hbm, idx_hbm, out_hbm, idx_v, rows_v, *sems):
    sem_g = sems[:_NBUF]
    sem_o = sems[_NBUF:]
    wid = lax.axis_index("s") * _NC + lax.axis_index("c")
    idx_row_base = wid * _IDX_ROWS_PER_W
    out_base = wid * _ROWS_PER_W

    pltpu.sync_copy(idx_hbm.at[pl.ds(idx_row_base, _IDX_ROWS_PER_W)], idx_v)

    def g_copy(c, b):
      return pltpu.make_async_copy(
          table_hbm.at[idx_v.at[c]], rows_v.at[b], sem_g[b])

    def o_copy(c, b):
      return pltpu.make_async_copy(
          rows_v.at[b],
          out_hbm.at[pl.ds(out_base + c * _CHUNK, _CHUNK)],
          sem_o[b],
      )

    for b in range(_NBUF):
      g_copy(b, b).start()

    def super_body(s, carry):
      c0 = s * _NBUF
      for b in range(_NBUF):
        c = c0 + b
        g_copy(c, b).wait()
        o_copy(c, b).start()
        pb = (b - 1) % _NBUF
        cprev = c - 1
        nxt = cprev + _NBUF

        @pl.when(jnp.logical_and(cprev >= 0, nxt < _N_CHUNKS))
        def _():
          o_copy(cprev, pb).wait()
          for cp in [g_copy(nxt, pb)]:
            cp.start()

      return carry

    lax.fori_loop(0, _N_CHUNKS // _NBUF, super_body, 0, unroll=False)

    for b in range(_NBUF):
      o_copy(_N_CHUNKS - _NBUF + b, b).wait()

  return gather_kernel


_transpose = _make_transpose()
_gather = _make_gather()


def kernel(input_, weight):
  idx = input_.reshape(-1).astype(jnp.int32).reshape(_N_IDX // _IDXW, _IDXW)
  tail = weight[_NBLK_FULL * _BLK:].reshape(32, 128)
  table2 = _transpose(weight.T, tail)
  table = table2.reshape(_VOCAB, _DIM)
  out = _gather(table, idx)
  return out.reshape(_B_ROWS, _SEQ, _DIM)
